# Initial kernel scaffold; baseline (speedup 1.0000x reference)
import functools

import jax
import jax.numpy as jnp
from jax import lax
from jax.experimental import pallas as pl
from jax.experimental.pallas import tpu as pltpu
from jax.experimental.pallas import tpu_sc as plsc

NC = 2
NS = 16
NW = NC * NS
LANES = 16
NORM = 100.0


def _precompute_body(h_ref, wa_ref, wb_ref, a_ref, b_ref):
    h = h_ref[...]
    a_ref[...] = jnp.dot(h, wa_ref[...], preferred_element_type=jnp.float32)
    b_ref[...] = jnp.dot(h, wb_ref[...], preferred_element_type=jnp.float32)


def _make_precompute(N, H, BN):
    return pl.pallas_call(
        _precompute_body,
        grid=(N // BN,),
        in_specs=[
            pl.BlockSpec((BN, H), lambda i: (i, 0)),
            pl.BlockSpec((H, H), lambda i: (0, 0)),
            pl.BlockSpec((H, H), lambda i: (0, 0)),
        ],
        out_specs=[
            pl.BlockSpec((BN, H), lambda i: (i, 0)),
            pl.BlockSpec((BN, H), lambda i: (i, 0)),
        ],
        out_shape=[
            jax.ShapeDtypeStruct((N, H), jnp.float32),
            jax.ShapeDtypeStruct((N, H), jnp.float32),
        ],
    )


def _make_gather(N, E, H, chunk):
    epw = E // NW
    nch = epw // chunk
    mesh = plsc.VectorSubcoreMesh(
        core_axis_name="c", subcore_axis_name="s",
        num_cores=NC, num_subcores=NS)

    @functools.partial(
        pl.kernel,
        out_type=jax.ShapeDtypeStruct((E, H), jnp.float32),
        mesh=mesh,
        scratch_types=[
            pltpu.VMEM((epw,), jnp.int32),
            pltpu.VMEM((epw,), jnp.int32),
            pltpu.VMEM((chunk, H), jnp.float32),
            pltpu.VMEM((chunk, H), jnp.float32),
            pltpu.SemaphoreType.DMA,
            pltpu.SemaphoreType.DMA,
        ],
    )
    def gather(row_hbm, col_hbm, a_hbm, b_hbm, g_hbm,
               idxr, idxc, bufa, bufb, sema, semb):
        wid = lax.axis_index("s") * NC + lax.axis_index("c")
        base = wid * epw
        pltpu.sync_copy(row_hbm.at[pl.ds(base, epw)], idxr)
        pltpu.sync_copy(col_hbm.at[pl.ds(base, epw)], idxc)

        def body(i, carry):
            off = i * chunk
            ca = pltpu.async_copy(a_hbm.at[idxr.at[pl.ds(off, chunk)]], bufa, sema)
            cb = pltpu.async_copy(b_hbm.at[idxc.at[pl.ds(off, chunk)]], bufb, semb)
            ca.wait()
            cb.wait()

            def add_row(j, c2):
                for k in range(H // LANES):
                    sl = pl.ds(k * LANES, LANES)
                    bufa[j, sl] = bufa[j, sl] + bufb[j, sl]
                return c2

            lax.fori_loop(0, chunk, add_row, 0)
            pltpu.sync_copy(bufa, g_hbm.at[pl.ds(base + off, chunk)])
            return carry

        lax.fori_loop(0, nch, body, 0)

    return gather


def _edge_mlp_body(g_ref, ea_ref, cd_ref, w1c_ref, b1_ref, w2t_ref, b2_ref,
                   w3_ref, out_ref):
    x1 = g_ref[...] + ea_ref[...] * w1c_ref[...] + b1_ref[...]
    x1 = x1 * jax.nn.sigmoid(x1)
    x2 = jnp.dot(x1, w2t_ref[...], preferred_element_type=jnp.float32)
    x2 = x2 + b2_ref[...]
    x2 = x2 * jax.nn.sigmoid(x2)
    m = lax.dot_general(w3_ref[...], x2, (((1,), (1,)), ((), ())),
                        preferred_element_type=jnp.float32)
    out_ref[...] = cd_ref[...] * m


def _make_edge_mlp(E, H, D, BE):
    return pl.pallas_call(
        _edge_mlp_body,
        grid=(E // BE,),
        in_specs=[
            pl.BlockSpec((BE, H), lambda i: (i, 0)),
            pl.BlockSpec((BE, D), lambda i: (i, 0)),
            pl.BlockSpec((3, BE), lambda i: (0, i)),
            pl.BlockSpec((D, H), lambda i: (0, 0)),
            pl.BlockSpec((1, H), lambda i: (0, 0)),
            pl.BlockSpec((H, H), lambda i: (0, 0)),
            pl.BlockSpec((1, H), lambda i: (0, 0)),
            pl.BlockSpec((1, H), lambda i: (0, 0)),
        ],
        out_specs=pl.BlockSpec((3, BE), lambda i: (0, i)),
        out_shape=jax.ShapeDtypeStruct((3, E), jnp.float32),
    )


def _make_scatter(N, E):
    epw = E // NW
    ngrp = epw // LANES
    nzero = N // LANES
    mesh = plsc.VectorSubcoreMesh(
        core_axis_name="c", subcore_axis_name="s",
        num_cores=NC, num_subcores=NS)

    @functools.partial(
        pl.kernel,
        out_type=jax.ShapeDtypeStruct((NW, 3, N), jnp.float32),
        mesh=mesh,
        scratch_types=[
            pltpu.VMEM((epw,), jnp.int32),
            pltpu.VMEM((3, epw), jnp.float32),
            pltpu.VMEM((N,), jnp.float32),
            pltpu.VMEM((N,), jnp.float32),
            pltpu.VMEM((N,), jnp.float32),
        ],
    )
    def scatter(row_hbm, trans_hbm, part_hbm, idxv, tv, ax, ay, az):
        wid = lax.axis_index("s") * NC + lax.axis_index("c")
        base = wid * epw
        pltpu.sync_copy(row_hbm.at[pl.ds(base, epw)], idxv)
        pltpu.sync_copy(trans_hbm.at[0, pl.ds(base, epw)], tv.at[0])
        pltpu.sync_copy(trans_hbm.at[1, pl.ds(base, epw)], tv.at[1])
        pltpu.sync_copy(trans_hbm.at[2, pl.ds(base, epw)], tv.at[2])

        zeros = jnp.zeros((LANES,), jnp.float32)

        def zbody(i, carry):
            sl = pl.ds(i * LANES, LANES)
            ax[sl] = zeros
            ay[sl] = zeros
            az[sl] = zeros
            return carry

        lax.fori_loop(0, nzero, zbody, 0)

        def sbody(g, carry):
            sl = pl.ds(g * LANES, LANES)
            idx = idxv[sl]
            plsc.addupdate_scatter(ax, [idx], tv[0, sl])
            plsc.addupdate_scatter(ay, [idx], tv[1, sl])
            plsc.addupdate_scatter(az, [idx], tv[2, sl])
            return carry

        lax.fori_loop(0, ngrp, sbody, 0)
        pltpu.sync_copy(ax, part_hbm.at[wid, 0])
        pltpu.sync_copy(ay, part_hbm.at[wid, 1])
        pltpu.sync_copy(az, part_hbm.at[wid, 2])

    return scatter


def _combine_body(part_ref, coordt_ref, out_ref):
    s = jnp.sum(part_ref[...], axis=0)
    out_ref[...] = coordt_ref[...] + s * (1.0 / NORM)


def _make_combine(N):
    return pl.pallas_call(
        _combine_body,
        grid=(1,),
        in_specs=[
            pl.BlockSpec((NW, 3, N), lambda i: (0, 0, 0)),
            pl.BlockSpec((3, N), lambda i: (0, 0)),
        ],
        out_specs=pl.BlockSpec((3, N), lambda i: (0, 0)),
        out_shape=jax.ShapeDtypeStruct((3, N), jnp.float32),
    )


def kernel(h, coord, coord_diff, edge_attr, W1, b1, W2, b2, W3, edge_index):
    N, H = h.shape
    E = edge_index.shape[1]
    D = edge_attr.shape[1]

    wa = W1[:, :H].T
    wb = W1[:, H:2 * H].T
    w1c = W1[:, 2 * H:].T
    row = edge_index[0]
    col = edge_index[1]

    A, B = _make_precompute(N, H, 2000)(h, wa, wb)
    G = _make_gather(N, E, H, 80)(row, col, A, B)
    trans = _make_edge_mlp(E, H, D, 2560)(
        G, edge_attr, coord_diff.T, w1c, b1.reshape(1, H), W2.T,
        b2.reshape(1, H), W3)
    parts = _make_scatter(N, E)(row, trans)
    outT = _make_combine(N)(parts, coord.T)
    return outT.T


# speedup vs baseline: 4.8963x; 4.8963x over previous
import functools

import jax
import jax.numpy as jnp
from jax import lax
from jax.experimental import pallas as pl
from jax.experimental.pallas import tpu as pltpu
from jax.experimental.pallas import tpu_sc as plsc

NC = 2
NS = 16
NW = NC * NS
LANES = 16
NORM = 100.0

_SC_PARAMS = pltpu.CompilerParams(needs_layout_passes=False)


def _precompute_body(h_ref, wa_ref, wb_ref, a_ref, b_ref):
    h = h_ref[...]
    a_ref[...] = jnp.dot(h, wa_ref[...], preferred_element_type=jnp.float32)
    b_ref[...] = jnp.dot(h, wb_ref[...], preferred_element_type=jnp.float32)


def _make_precompute(N, H, BN):
    return pl.pallas_call(
        _precompute_body,
        grid=(N // BN,),
        in_specs=[
            pl.BlockSpec((BN, H), lambda i: (i, 0)),
            pl.BlockSpec((H, H), lambda i: (0, 0)),
            pl.BlockSpec((H, H), lambda i: (0, 0)),
        ],
        out_specs=[
            pl.BlockSpec((BN, H), lambda i: (i, 0)),
            pl.BlockSpec((BN, H), lambda i: (i, 0)),
        ],
        out_shape=[
            jax.ShapeDtypeStruct((N, H), jnp.float32),
            jax.ShapeDtypeStruct((N, H), jnp.float32),
        ],
    )


def _make_gather(N, E, H, chunk):
    epw = E // NW
    nch = epw // chunk
    mesh = plsc.VectorSubcoreMesh(
        core_axis_name="c", subcore_axis_name="s",
        num_cores=NC, num_subcores=NS)

    @functools.partial(
        pl.kernel,
        out_type=jax.ShapeDtypeStruct((E, H), jnp.float32),
        mesh=mesh,
        scratch_types=[
            pltpu.VMEM((epw,), jnp.int32),
            pltpu.VMEM((epw,), jnp.int32),
            pltpu.VMEM((chunk, H), jnp.float32),
            pltpu.VMEM((chunk, H), jnp.float32),
            pltpu.SemaphoreType.DMA,
            pltpu.SemaphoreType.DMA,
        ],
        compiler_params=_SC_PARAMS,
    )
    def gather(row_hbm, col_hbm, a_hbm, b_hbm, g_hbm,
               idxr, idxc, bufa, bufb, sema, semb):
        wid = lax.axis_index("s") * NC + lax.axis_index("c")
        base = wid * epw
        pltpu.sync_copy(row_hbm.at[pl.ds(base, epw)], idxr)
        pltpu.sync_copy(col_hbm.at[pl.ds(base, epw)], idxc)

        def body(i, carry):
            off = i * chunk
            ca = pltpu.async_copy(a_hbm.at[idxr.at[pl.ds(off, chunk)]], bufa, sema)
            cb = pltpu.async_copy(b_hbm.at[idxc.at[pl.ds(off, chunk)]], bufb, semb)
            ca.wait()
            cb.wait()

            def add_row(j, c2):
                for k in range(H // LANES):
                    sl = pl.ds(k * LANES, LANES)
                    bufa[j, sl] = bufa[j, sl] + bufb[j, sl]
                return c2

            lax.fori_loop(0, chunk, add_row, 0)
            pltpu.sync_copy(bufa, g_hbm.at[pl.ds(base + off, chunk)])
            return carry

        lax.fori_loop(0, nch, body, 0)

    return gather


def _edge_mlp_body(g_ref, ea_ref, cdx_ref, cdy_ref, cdz_ref, w1c_ref, b1_ref,
                   w2t_ref, b2_ref, w3_ref, tx_ref, ty_ref, tz_ref):
    x1 = g_ref[...] + ea_ref[...] * w1c_ref[...] + b1_ref[...]
    x1 = x1 * jax.nn.sigmoid(x1)
    x2 = jnp.dot(x1, w2t_ref[...], preferred_element_type=jnp.float32)
    x2 = x2 + b2_ref[...]
    x2 = x2 * jax.nn.sigmoid(x2)
    m = lax.dot_general(w3_ref[...], x2, (((1,), (1,)), ((), ())),
                        preferred_element_type=jnp.float32)
    tx_ref[...] = cdx_ref[...] * m
    ty_ref[...] = cdy_ref[...] * m
    tz_ref[...] = cdz_ref[...] * m


def _make_edge_mlp(E, H, D, BE):
    row_spec = pl.BlockSpec((1, BE), lambda i: (0, i))
    return pl.pallas_call(
        _edge_mlp_body,
        grid=(E // BE,),
        in_specs=[
            pl.BlockSpec((BE, H), lambda i: (i, 0)),
            pl.BlockSpec((BE, D), lambda i: (i, 0)),
            row_spec,
            row_spec,
            row_spec,
            pl.BlockSpec((D, H), lambda i: (0, 0)),
            pl.BlockSpec((1, H), lambda i: (0, 0)),
            pl.BlockSpec((H, H), lambda i: (0, 0)),
            pl.BlockSpec((1, H), lambda i: (0, 0)),
            pl.BlockSpec((1, H), lambda i: (0, 0)),
        ],
        out_specs=[row_spec, row_spec, row_spec],
        out_shape=[jax.ShapeDtypeStruct((1, E), jnp.float32)] * 3,
    )


def _make_scatter(N, E):
    epw = E // NW
    ngrp = epw // LANES
    nzero = N // LANES
    mesh = plsc.VectorSubcoreMesh(
        core_axis_name="c", subcore_axis_name="s",
        num_cores=NC, num_subcores=NS)

    @functools.partial(
        pl.kernel,
        out_type=[jax.ShapeDtypeStruct((NW * N,), jnp.float32)] * 3,
        mesh=mesh,
        scratch_types=[
            pltpu.VMEM((epw,), jnp.int32),
            pltpu.VMEM((epw,), jnp.float32),
            pltpu.VMEM((epw,), jnp.float32),
            pltpu.VMEM((epw,), jnp.float32),
            pltpu.VMEM((N,), jnp.float32),
            pltpu.VMEM((N,), jnp.float32),
            pltpu.VMEM((N,), jnp.float32),
        ],
        compiler_params=_SC_PARAMS,
    )
    def scatter(row_hbm, tx_hbm, ty_hbm, tz_hbm, px_hbm, py_hbm, pz_hbm,
                idxv, tvx, tvy, tvz, ax, ay, az):
        wid = lax.axis_index("s") * NC + lax.axis_index("c")
        base = wid * epw
        pltpu.sync_copy(row_hbm.at[pl.ds(base, epw)], idxv)
        pltpu.sync_copy(tx_hbm.at[pl.ds(base, epw)], tvx)
        pltpu.sync_copy(ty_hbm.at[pl.ds(base, epw)], tvy)
        pltpu.sync_copy(tz_hbm.at[pl.ds(base, epw)], tvz)

        zeros = jnp.zeros((LANES,), jnp.float32)

        def zbody(i, carry):
            sl = pl.ds(i * LANES, LANES)
            ax[sl] = zeros
            ay[sl] = zeros
            az[sl] = zeros
            return carry

        lax.fori_loop(0, nzero, zbody, 0)

        def sbody(g, carry):
            sl = pl.ds(g * LANES, LANES)
            idx = idxv[sl]
            plsc.addupdate_scatter(ax, [idx], tvx[sl])
            plsc.addupdate_scatter(ay, [idx], tvy[sl])
            plsc.addupdate_scatter(az, [idx], tvz[sl])
            return carry

        lax.fori_loop(0, ngrp, sbody, 0)
        pltpu.sync_copy(ax, px_hbm.at[pl.ds(wid * N, N)])
        pltpu.sync_copy(ay, py_hbm.at[pl.ds(wid * N, N)])
        pltpu.sync_copy(az, pz_hbm.at[pl.ds(wid * N, N)])

    return scatter


def _combine_body(px_ref, py_ref, pz_ref, cx_ref, cy_ref, cz_ref,
                  ox_ref, oy_ref, oz_ref):
    scale = 1.0 / NORM
    ox_ref[...] = cx_ref[...] + jnp.sum(px_ref[...], axis=0,
                                        keepdims=True) * scale
    oy_ref[...] = cy_ref[...] + jnp.sum(py_ref[...], axis=0,
                                        keepdims=True) * scale
    oz_ref[...] = cz_ref[...] + jnp.sum(pz_ref[...], axis=0,
                                        keepdims=True) * scale


def _make_combine(N):
    part_spec = pl.BlockSpec((NW, N), lambda i: (0, 0))
    row_spec = pl.BlockSpec((1, N), lambda i: (0, 0))
    return pl.pallas_call(
        _combine_body,
        grid=(1,),
        in_specs=[part_spec, part_spec, part_spec, row_spec, row_spec,
                  row_spec],
        out_specs=[row_spec, row_spec, row_spec],
        out_shape=[jax.ShapeDtypeStruct((1, N), jnp.float32)] * 3,
    )


def kernel(h, coord, coord_diff, edge_attr, W1, b1, W2, b2, W3, edge_index):
    N, H = h.shape
    E = edge_index.shape[1]
    D = edge_attr.shape[1]

    wa = W1[:, :H].T
    wb = W1[:, H:2 * H].T
    w1c = W1[:, 2 * H:].T
    row = edge_index[0]
    col = edge_index[1]
    cdT = coord_diff.T

    A, B = _make_precompute(N, H, 2000)(h, wa, wb)
    G = _make_gather(N, E, H, 80)(row, col, A, B)
    tx, ty, tz = _make_edge_mlp(E, H, D, 2560)(
        G, edge_attr, cdT[0].reshape(1, E), cdT[1].reshape(1, E),
        cdT[2].reshape(1, E), w1c, b1.reshape(1, H), W2.T,
        b2.reshape(1, H), W3)
    px, py, pz = _make_scatter(N, E)(row, tx.reshape(E), ty.reshape(E),
                                     tz.reshape(E))
    coordT = coord.T
    ox, oy, oz = _make_combine(N)(px.reshape(NW, N), py.reshape(NW, N),
                                  pz.reshape(NW, N),
                                  coordT[0].reshape(1, N),
                                  coordT[1].reshape(1, N),
                                  coordT[2].reshape(1, N))
    return jnp.concatenate([ox, oy, oz], axis=0).T


# speedup vs baseline: 6.0060x; 1.2267x over previous
import functools

import jax
import jax.numpy as jnp
from jax import lax
from jax.experimental import pallas as pl
from jax.experimental.pallas import tpu as pltpu
from jax.experimental.pallas import tpu_sc as plsc

NC = 2
NS = 16
NW = NC * NS
LANES = 16
NORM = 100.0

_SC_PARAMS = pltpu.CompilerParams(needs_layout_passes=False)


def _precompute_body(h_ref, wa_ref, wb_ref, a_ref, b_ref):
    h = h_ref[...]
    a_ref[...] = jnp.dot(h, wa_ref[...], preferred_element_type=jnp.float32)
    b_ref[...] = jnp.dot(h, wb_ref[...], preferred_element_type=jnp.float32)


def _make_precompute(N, H, BN):
    return pl.pallas_call(
        _precompute_body,
        grid=(N // BN,),
        in_specs=[
            pl.BlockSpec((BN, H), lambda i: (i, 0)),
            pl.BlockSpec((H, H), lambda i: (0, 0)),
            pl.BlockSpec((H, H), lambda i: (0, 0)),
        ],
        out_specs=[
            pl.BlockSpec((BN, H), lambda i: (i, 0)),
            pl.BlockSpec((BN, H), lambda i: (i, 0)),
        ],
        out_shape=[
            jax.ShapeDtypeStruct((N, H), jnp.float32),
            jax.ShapeDtypeStruct((N, H), jnp.float32),
        ],
    )


def _make_gather(N, E, H, slot):
    epw = E // NW
    nslot = epw // slot
    npair = nslot // 2
    subs = []
    off = 0
    while off < slot:
        sz = min(80, slot - off)
        subs.append((off, sz))
        off += sz
    mesh = plsc.VectorSubcoreMesh(
        core_axis_name="c", subcore_axis_name="s",
        num_cores=NC, num_subcores=NS)

    @functools.partial(
        pl.kernel,
        out_type=jax.ShapeDtypeStruct((E, H), jnp.float32),
        mesh=mesh,
        scratch_types=[
            pltpu.VMEM((epw,), jnp.int32),
            pltpu.VMEM((epw,), jnp.int32),
            [pltpu.VMEM((slot, H), jnp.float32)] * 2,
            [pltpu.VMEM((slot, H), jnp.float32)] * 2,
            [pltpu.SemaphoreType.DMA] * 2,
            [pltpu.SemaphoreType.DMA] * 2,
            [pltpu.SemaphoreType.DMA] * 2,
        ],
        compiler_params=_SC_PARAMS,
    )
    def gather(row_hbm, col_hbm, a_hbm, b_hbm, g_hbm,
               idxr, idxc, bufa, bufb, sema, semb, semo):
        wid = lax.axis_index("s") * NC + lax.axis_index("c")
        base = wid * epw
        pltpu.sync_copy(row_hbm.at[pl.ds(base, epw)], idxr)
        pltpu.sync_copy(col_hbm.at[pl.ds(base, epw)], idxc)

        def issue(c, k):
            coff = c * slot
            for (o, sz) in subs:
                pltpu.async_copy(
                    a_hbm.at[idxr.at[pl.ds(coff + o, sz)]],
                    bufa[k].at[pl.ds(o, sz)], sema[k])
                pltpu.async_copy(
                    b_hbm.at[idxc.at[pl.ds(coff + o, sz)]],
                    bufb[k].at[pl.ds(o, sz)], semb[k])

        def wait_gathers(k):
            pltpu.make_async_copy(a_hbm.at[pl.ds(0, slot)], bufa[k],
                                  sema[k]).wait()
            pltpu.make_async_copy(b_hbm.at[pl.ds(0, slot)], bufb[k],
                                  semb[k]).wait()

        def add(k):
            ba, bb = bufa[k], bufb[k]

            def add_row(j, c2):
                for kk in range(H // LANES):
                    sl = pl.ds(kk * LANES, LANES)
                    ba[j, sl] = ba[j, sl] + bb[j, sl]
                return c2

            lax.fori_loop(0, slot, add_row, 0)

        def process(c, k, refill):
            wait_gathers(k)
            add(k)
            wr = pltpu.async_copy(bufa[k], g_hbm.at[pl.ds(base + c * slot, slot)],
                                  semo[k])

            @pl.when(refill)
            def _():
                wr.wait()
                issue(c + 2, k)

        issue(0, 0)
        issue(1, 1)

        def body(i, carry):
            refill = i < npair - 1
            process(2 * i, 0, refill)
            process(2 * i + 1, 1, refill)
            return carry

        lax.fori_loop(0, npair, body, 0)
        pltpu.make_async_copy(g_hbm.at[pl.ds(0, slot)], bufa[0], semo[0]).wait()
        pltpu.make_async_copy(g_hbm.at[pl.ds(0, slot)], bufa[1], semo[1]).wait()

    return gather


def _edge_mlp_body(g_ref, ea_ref, cdx_ref, cdy_ref, cdz_ref, w1c_ref, b1_ref,
                   w2t_ref, b2_ref, w3_ref, tx_ref, ty_ref, tz_ref):
    x1 = g_ref[...] + ea_ref[...] * w1c_ref[...] + b1_ref[...]
    x1 = x1 * jax.nn.sigmoid(x1)
    x2 = jnp.dot(x1, w2t_ref[...], preferred_element_type=jnp.float32)
    x2 = x2 + b2_ref[...]
    x2 = x2 * jax.nn.sigmoid(x2)
    m = lax.dot_general(w3_ref[...], x2, (((1,), (1,)), ((), ())),
                        preferred_element_type=jnp.float32)
    tx_ref[...] = cdx_ref[...] * m
    ty_ref[...] = cdy_ref[...] * m
    tz_ref[...] = cdz_ref[...] * m


def _make_edge_mlp(E, H, D, BE):
    row_spec = pl.BlockSpec((1, BE), lambda i: (0, i))
    return pl.pallas_call(
        _edge_mlp_body,
        grid=(E // BE,),
        in_specs=[
            pl.BlockSpec((BE, H), lambda i: (i, 0)),
            pl.BlockSpec((BE, D), lambda i: (i, 0)),
            row_spec,
            row_spec,
            row_spec,
            pl.BlockSpec((D, H), lambda i: (0, 0)),
            pl.BlockSpec((1, H), lambda i: (0, 0)),
            pl.BlockSpec((H, H), lambda i: (0, 0)),
            pl.BlockSpec((1, H), lambda i: (0, 0)),
            pl.BlockSpec((1, H), lambda i: (0, 0)),
        ],
        out_specs=[row_spec, row_spec, row_spec],
        out_shape=[jax.ShapeDtypeStruct((1, E), jnp.float32)] * 3,
    )


def _make_scatter(N, E):
    epw = E // NW
    ngrp = epw // LANES
    nzero = N // LANES
    mesh = plsc.VectorSubcoreMesh(
        core_axis_name="c", subcore_axis_name="s",
        num_cores=NC, num_subcores=NS)

    @functools.partial(
        pl.kernel,
        out_type=[jax.ShapeDtypeStruct((NW * N,), jnp.float32)] * 3,
        mesh=mesh,
        scratch_types=[
            pltpu.VMEM((epw,), jnp.int32),
            pltpu.VMEM((epw,), jnp.float32),
            pltpu.VMEM((epw,), jnp.float32),
            pltpu.VMEM((epw,), jnp.float32),
            pltpu.VMEM((N,), jnp.float32),
            pltpu.VMEM((N,), jnp.float32),
            pltpu.VMEM((N,), jnp.float32),
        ],
        compiler_params=_SC_PARAMS,
    )
    def scatter(row_hbm, tx_hbm, ty_hbm, tz_hbm, px_hbm, py_hbm, pz_hbm,
                idxv, tvx, tvy, tvz, ax, ay, az):
        wid = lax.axis_index("s") * NC + lax.axis_index("c")
        base = wid * epw
        pltpu.sync_copy(row_hbm.at[pl.ds(base, epw)], idxv)
        pltpu.sync_copy(tx_hbm.at[pl.ds(base, epw)], tvx)
        pltpu.sync_copy(ty_hbm.at[pl.ds(base, epw)], tvy)
        pltpu.sync_copy(tz_hbm.at[pl.ds(base, epw)], tvz)

        zeros = jnp.zeros((LANES,), jnp.float32)

        def zbody(i, carry):
            sl = pl.ds(i * LANES, LANES)
            ax[sl] = zeros
            ay[sl] = zeros
            az[sl] = zeros
            return carry

        lax.fori_loop(0, nzero, zbody, 0)

        def sbody(g, carry):
            sl = pl.ds(g * LANES, LANES)
            idx = idxv[sl]
            plsc.addupdate_scatter(ax, [idx], tvx[sl])
            plsc.addupdate_scatter(ay, [idx], tvy[sl])
            plsc.addupdate_scatter(az, [idx], tvz[sl])
            return carry

        lax.fori_loop(0, ngrp, sbody, 0)
        pltpu.sync_copy(ax, px_hbm.at[pl.ds(wid * N, N)])
        pltpu.sync_copy(ay, py_hbm.at[pl.ds(wid * N, N)])
        pltpu.sync_copy(az, pz_hbm.at[pl.ds(wid * N, N)])

    return scatter


def _combine_body(px_ref, py_ref, pz_ref, cx_ref, cy_ref, cz_ref,
                  ox_ref, oy_ref, oz_ref):
    scale = 1.0 / NORM
    ox_ref[...] = cx_ref[...] + jnp.sum(px_ref[...], axis=0,
                                        keepdims=True) * scale
    oy_ref[...] = cy_ref[...] + jnp.sum(py_ref[...], axis=0,
                                        keepdims=True) * scale
    oz_ref[...] = cz_ref[...] + jnp.sum(pz_ref[...], axis=0,
                                        keepdims=True) * scale


def _make_combine(N):
    part_spec = pl.BlockSpec((NW, N), lambda i: (0, 0))
    row_spec = pl.BlockSpec((1, N), lambda i: (0, 0))
    return pl.pallas_call(
        _combine_body,
        grid=(1,),
        in_specs=[part_spec, part_spec, part_spec, row_spec, row_spec,
                  row_spec],
        out_specs=[row_spec, row_spec, row_spec],
        out_shape=[jax.ShapeDtypeStruct((1, N), jnp.float32)] * 3,
    )


def kernel(h, coord, coord_diff, edge_attr, W1, b1, W2, b2, W3, edge_index):
    N, H = h.shape
    E = edge_index.shape[1]
    D = edge_attr.shape[1]

    wa = W1[:, :H].T
    wb = W1[:, H:2 * H].T
    w1c = W1[:, 2 * H:].T
    row = edge_index[0]
    col = edge_index[1]
    cdT = coord_diff.T

    A, B = _make_precompute(N, H, 2000)(h, wa, wb)
    G = _make_gather(N, E, H, 200)(row, col, A, B)
    tx, ty, tz = _make_edge_mlp(E, H, D, 2560)(
        G, edge_attr, cdT[0].reshape(1, E), cdT[1].reshape(1, E),
        cdT[2].reshape(1, E), w1c, b1.reshape(1, H), W2.T,
        b2.reshape(1, H), W3)
    px, py, pz = _make_scatter(N, E)(row, tx.reshape(E), ty.reshape(E),
                                     tz.reshape(E))
    coordT = coord.T
    ox, oy, oz = _make_combine(N)(px.reshape(NW, N), py.reshape(NW, N),
                                  pz.reshape(NW, N),
                                  coordT[0].reshape(1, N),
                                  coordT[1].reshape(1, N),
                                  coordT[2].reshape(1, N))
    return jnp.concatenate([ox, oy, oz], axis=0).T
